# baseline (speedup 1.0000x reference)
import functools

import jax
import jax.numpy as jnp
from jax import lax
from jax.experimental import pallas as pl
from jax.experimental.pallas import tpu as pltpu
from jax.experimental.pallas import tpu_sc as plsc

_NC = 2
_NS = 16


def _build_sc_lookup(B, D):
    nw = _NC * _NS
    b_per_w = B // nw
    half = b_per_w // 2
    assert B % (8 * nw) == 0 and D % 16 == 0

    mesh = plsc.VectorSubcoreMesh(core_axis_name="c", subcore_axis_name="s")

    @functools.partial(
        pl.kernel,
        mesh=mesh,
        compiler_params=pltpu.CompilerParams(use_tc_tiling_on_sc=False),
        out_type=(
            jax.ShapeDtypeStruct((B, D), jnp.float32),
            jax.ShapeDtypeStruct((B, D), jnp.float32),
        ),
        scratch_types=[
            pltpu.VMEM((b_per_w,), jnp.int32),
            pltpu.VMEM((b_per_w,), jnp.int32),
            pltpu.VMEM((half, D), jnp.float32),
            pltpu.VMEM((half, D), jnp.float32),
            pltpu.SemaphoreType.DMA,
            pltpu.SemaphoreType.DMA,
        ],
    )
    def _emb(uid_hbm, iid_hbm, utab_hbm, itab_hbm, u_out, i_out,
             uidx_v, iidx_v, rows_a, rows_b, sem_a, sem_b):
        wid = lax.axis_index("s") * _NC + lax.axis_index("c")
        base = wid * b_per_w
        pltpu.sync_copy(uid_hbm.at[pl.ds(base, b_per_w)], uidx_v)
        pltpu.sync_copy(iid_hbm.at[pl.ds(base, b_per_w)], iidx_v)
        ca = pltpu.async_copy(utab_hbm.at[uidx_v.at[pl.ds(0, half)]],
                              rows_a, sem_a)
        cb = pltpu.async_copy(utab_hbm.at[uidx_v.at[pl.ds(half, half)]],
                              rows_b, sem_b)
        ca.wait()
        pltpu.sync_copy(rows_a, u_out.at[pl.ds(base, half)])
        ca = pltpu.async_copy(itab_hbm.at[iidx_v.at[pl.ds(0, half)]],
                              rows_a, sem_a)
        cb.wait()
        pltpu.sync_copy(rows_b, u_out.at[pl.ds(base + half, half)])
        cb = pltpu.async_copy(itab_hbm.at[iidx_v.at[pl.ds(half, half)]],
                              rows_b, sem_b)
        ca.wait()
        pltpu.sync_copy(rows_a, i_out.at[pl.ds(base, half)])
        cb.wait()
        pltpu.sync_copy(rows_b, i_out.at[pl.ds(base + half, half)])

    return _emb


def kernel(user_id, item_id, user_emb, item_emb):
    B = user_id.shape[0]
    D = user_emb.shape[1]
    emb = _build_sc_lookup(B, D)
    return emb(user_id.astype(jnp.int32), item_id.astype(jnp.int32),
               user_emb, item_emb)
